# Initial kernel scaffold; baseline (speedup 1.0000x reference)
import jax
import jax.numpy as jnp
from jax.experimental import pallas as pl


def kernel(x, edge_index, W1_lin, b1_lin, W1_agg, b1_agg, W2_lin, b2_lin, W2_agg, b2_agg, W_out, b_out):
    raise NotImplementedError("write your pallas kernel here")


# speedup vs baseline: 24.0462x; 24.0462x over previous
import functools

import jax
import jax.numpy as jnp
from jax import lax
from jax.experimental import pallas as pl
from jax.experimental.pallas import tpu as pltpu
from jax.experimental.pallas import tpu_sc as plsc

N_NODES = 100000
NPAD = 100352
RPT = NPAD // 16
F = 16

NW = 32
CHUNK = 128
K1 = 8
G1 = 98
EPW = G1 * K1 * CHUNK
EPAD = NW * EPW
TOTAL_CHUNKS = EPAD // CHUNK

BN = 3136


def _make_edge_aggregate(with_cnt):
  mesh = plsc.VectorSubcoreMesh(core_axis_name="c", subcore_axis_name="s",
                                num_cores=2, num_subcores=16)

  out_type = [jax.ShapeDtypeStruct((2, NPAD, F), jnp.float32)]
  scratch = [
      pltpu.VMEM_SHARED((NPAD, F), jnp.float32),
      pltpu.VMEM((K1, CHUNK), jnp.int32),
      pltpu.VMEM((K1, CHUNK), jnp.int32),
      pltpu.VMEM((K1, CHUNK, F), jnp.float32),
      pltpu.SemaphoreType.DMA,
  ]
  if with_cnt:
    out_type.append(jax.ShapeDtypeStruct((2, NPAD), jnp.float32))
    scratch += [
        pltpu.VMEM_SHARED((NPAD,), jnp.float32),
        pltpu.VMEM((CHUNK,), jnp.float32),
    ]

  def body(*refs):
    if with_cnt:
      (y_hbm, src_hbm, dst_hbm, z16_hbm, z1_hbm,
       out_hbm, cnt_hbm, acc_sh, sidx, didx, rows, sem, cnt_sh, ones_v) = refs
    else:
      (y_hbm, src_hbm, dst_hbm, z16_hbm,
       out_hbm, acc_sh, sidx, didx, rows, sem) = refs

    c = lax.axis_index("c")
    s = lax.axis_index("s")
    w = s * 2 + c
    row0 = s * RPT

    pltpu.sync_copy(z16_hbm.at[pl.ds(row0, RPT)], acc_sh.at[pl.ds(row0, RPT)])
    if with_cnt:
      pltpu.sync_copy(z1_hbm.at[pl.ds(row0, RPT)], cnt_sh.at[pl.ds(row0, RPT)])
      for i in range(CHUNK // 16):
        ones_v[pl.ds(i * 16, 16)] = jnp.full((16,), 1.0, jnp.float32)
    plsc.subcore_barrier()

    base = w * (G1 * K1)

    def group(g, carry):
      g0 = base + g * K1
      pltpu.sync_copy(src_hbm.at[pl.ds(g0, K1)], sidx)
      pltpu.sync_copy(dst_hbm.at[pl.ds(g0, K1)], didx)
      cps = [pltpu.async_copy(y_hbm.at[sidx.at[b]], rows.at[b], sem)
             for b in range(K1)]
      for cp in cps:
        cp.wait()
      for b in range(K1):
        pltpu.sync_copy(rows.at[b], acc_sh.at[didx.at[b]], add=True)
      if with_cnt:
        for b in range(K1):
          pltpu.sync_copy(ones_v, cnt_sh.at[didx.at[b]], add=True)
      return carry

    lax.fori_loop(0, G1, group, 0)
    plsc.subcore_barrier()

    pltpu.sync_copy(acc_sh.at[pl.ds(row0, RPT)],
                    out_hbm.at[c, pl.ds(row0, RPT)])
    if with_cnt:
      pltpu.sync_copy(cnt_sh.at[pl.ds(row0, RPT)],
                      cnt_hbm.at[c, pl.ds(row0, RPT)])

  return pl.kernel(body,
                   out_type=tuple(out_type) if with_cnt else out_type[0],
                   mesh=mesh, scratch_types=scratch,
                   compiler_params=pltpu.CompilerParams(
                       use_tc_tiling_on_sc=False))


_edge_agg_cnt = _make_edge_aggregate(True)
_edge_agg = _make_edge_aggregate(False)


def _lin_relu_body(x_ref, w_ref, b_ref, o_ref):
  o_ref[...] = jnp.maximum(
      jnp.dot(x_ref[...], w_ref[...], preferred_element_type=jnp.float32)
      + b_ref[...], 0.0)


def _lin_relu(x, wt, b):
  din = x.shape[1]
  return pl.pallas_call(
      _lin_relu_body,
      grid=(NPAD // BN,),
      in_specs=[pl.BlockSpec((BN, din), lambda i: (i, 0)),
                pl.BlockSpec((din, F), lambda i: (0, 0)),
                pl.BlockSpec((1, F), lambda i: (0, 0))],
      out_specs=pl.BlockSpec((BN, F), lambda i: (i, 0)),
      out_shape=jax.ShapeDtypeStruct((NPAD, F), jnp.float32),
  )(x, wt, b)


def _update_body(want_y2, want_out,
                 x_ref, s_ref, cnt_ref, wx_ref, wm_ref, ba_ref,
                 w2_ref, b2_ref, *out_refs):
  ssum = s_ref[0] + s_ref[1]
  cc = cnt_ref[0] + cnt_ref[1]
  mean = ssum / jnp.maximum(cc, 1.0)
  h = (jnp.dot(x_ref[...], wx_ref[...], preferred_element_type=jnp.float32)
       + jnp.dot(mean, wm_ref[...], preferred_element_type=jnp.float32)
       + ba_ref[...])
  h = jnp.maximum(h, 0.0)
  norm = jnp.sqrt(jnp.sum(h * h, axis=1, keepdims=True))
  h = h / jnp.maximum(norm, 1e-12)
  o = 0
  if want_y2:
    out_refs[o][...] = h
    o += 1
    out_refs[o][...] = jnp.maximum(
        jnp.dot(h, w2_ref[...], preferred_element_type=jnp.float32)
        + b2_ref[...], 0.0)
    o += 1
  if want_out:
    out_refs[o][...] = (
        jnp.dot(h, w2_ref[...], preferred_element_type=jnp.float32)
        + b2_ref[...])


def _sage_update(x, sums, cnt, w_agg, b_agg, w_next, b_next,
                 want_y2):
  din = x.shape[1]
  wx = w_agg.T[:din]
  wm = w_agg.T[din:]
  fo = w_next.shape[0]
  if want_y2:
    out_shape = (jax.ShapeDtypeStruct((NPAD, F), jnp.float32),
                 jax.ShapeDtypeStruct((NPAD, fo), jnp.float32))
    out_specs = (pl.BlockSpec((BN, F), lambda i: (i, 0)),
                 pl.BlockSpec((BN, fo), lambda i: (i, 0)))
  else:
    out_shape = jax.ShapeDtypeStruct((NPAD, fo), jnp.float32)
    out_specs = pl.BlockSpec((BN, fo), lambda i: (i, 0))
  return pl.pallas_call(
      functools.partial(_update_body, want_y2, not want_y2),
      grid=(NPAD // BN,),
      in_specs=[pl.BlockSpec((BN, din), lambda i: (i, 0)),
                pl.BlockSpec((2, BN, F), lambda i: (0, i, 0)),
                pl.BlockSpec((2, BN, 1), lambda i: (0, i, 0)),
                pl.BlockSpec((din, F), lambda i: (0, 0)),
                pl.BlockSpec((F, F), lambda i: (0, 0)),
                pl.BlockSpec((1, F), lambda i: (0, 0)),
                pl.BlockSpec((F, fo), lambda i: (0, 0)),
                pl.BlockSpec((1, fo), lambda i: (0, 0))],
      out_specs=out_specs,
      out_shape=out_shape,
  )(x, sums, cnt, wx, wm, b_agg[None, :], w_next.T, b_next[None, :])


def kernel(x, edge_index, W1_lin, b1_lin, W1_agg, b1_agg,
           W2_lin, b2_lin, W2_agg, b2_agg, W_out, b_out):
  n = x.shape[0]
  e = edge_index.shape[1]

  x_pad = jnp.pad(x.astype(jnp.float32), ((0, NPAD - n), (0, 0)))

  src = edge_index[0].astype(jnp.int32)
  dst = edge_index[1].astype(jnp.int32)
  pad_e = EPAD - e
  fill = jnp.full((pad_e,), n, jnp.int32)
  src_c = jnp.concatenate([src, fill]).reshape(TOTAL_CHUNKS, CHUNK)
  dst_c = jnp.concatenate([dst, fill]).reshape(TOTAL_CHUNKS, CHUNK)

  z16 = jnp.zeros((NPAD, F), jnp.float32)
  z1 = jnp.zeros((NPAD,), jnp.float32)

  y1 = _lin_relu(x_pad, W1_lin.T, b1_lin[None, :])
  s1, cnt = _edge_agg_cnt(y1, src_c, dst_c, z16, z1)
  cnt3 = cnt.reshape(2, NPAD, 1)
  h1, y2 = _sage_update(x_pad, s1, cnt3, W1_agg, b1_agg, W2_lin, b2_lin,
                        want_y2=True)

  s2 = _edge_agg(y2, src_c, dst_c, z16)
  out = _sage_update(h1, s2, cnt3, W2_agg, b2_agg, W_out, b_out,
                     want_y2=False)
  return out[:n]


# speedup vs baseline: 31.2288x; 1.2987x over previous
import functools

import jax
import jax.numpy as jnp
from jax import lax
from jax.experimental import pallas as pl
from jax.experimental.pallas import tpu as pltpu
from jax.experimental.pallas import tpu_sc as plsc

N_NODES = 100000
NPAD = 100352
RPT = NPAD // 16
F = 16

NW = 32
CHUNK = 128
K1 = 4
G1 = 196
EPW = G1 * K1 * CHUNK
EPAD = NW * EPW
TOTAL_CHUNKS = EPAD // CHUNK

BN = 3136


def _make_edge_aggregate(with_cnt):
  mesh = plsc.VectorSubcoreMesh(core_axis_name="c", subcore_axis_name="s",
                                num_cores=2, num_subcores=16)

  out_type = [jax.ShapeDtypeStruct((2, NPAD, F), jnp.float32)]
  scratch = [
      pltpu.VMEM_SHARED((NPAD, F), jnp.float32),
      pltpu.VMEM((2, K1, 2, CHUNK), jnp.int32),
      pltpu.VMEM((2, K1, CHUNK, F), jnp.float32),
      pltpu.SemaphoreType.DMA,
      pltpu.SemaphoreType.DMA,
      pltpu.SemaphoreType.DMA,
      pltpu.SemaphoreType.DMA,
  ]
  if with_cnt:
    out_type.append(jax.ShapeDtypeStruct((2, NPAD), jnp.float32))
    scratch += [
        pltpu.VMEM_SHARED((NPAD,), jnp.float32),
        pltpu.VMEM((CHUNK,), jnp.float32),
    ]

  def body(*refs):
    if with_cnt:
      (y_hbm, sd_hbm, z16_hbm, z1_hbm, out_hbm, cnt_hbm,
       acc_sh, sd, rows, gs0, gs1, ss0, ss1, cnt_sh, ones_v) = refs
    else:
      (y_hbm, sd_hbm, z16_hbm, out_hbm,
       acc_sh, sd, rows, gs0, gs1, ss0, ss1) = refs
    gsem = (gs0, gs1)
    ssem = (ss0, ss1)

    c = lax.axis_index("c")
    s = lax.axis_index("s")
    w = s * 2 + c
    row0 = s * RPT
    base = w * G1 * K1

    def stage(p, g):
      pltpu.sync_copy(sd_hbm.at[pl.ds(base + g * K1, K1)], sd.at[p])

    def fire_g(p):
      for b in range(K1):
        pltpu.async_copy(y_hbm.at[sd.at[p, b, 0]], rows.at[p, b], gsem[p])

    def wait_g(p):
      for b in range(K1):
        pltpu.make_async_copy(y_hbm.at[sd.at[p, b, 0]], rows.at[p, b],
                              gsem[p]).wait()

    def fire_s(p):
      for b in range(K1):
        pltpu.async_copy(rows.at[p, b], acc_sh.at[sd.at[p, b, 1]], ssem[p],
                         add=True)
      if with_cnt:
        for b in range(K1):
          pltpu.async_copy(ones_v, cnt_sh.at[sd.at[p, b, 1]], ssem[p],
                           add=True)

    def wait_s(p):
      for b in range(K1):
        pltpu.make_async_copy(rows.at[p, b], acc_sh.at[sd.at[p, b, 1]],
                              ssem[p]).wait()
      if with_cnt:
        for b in range(K1):
          pltpu.make_async_copy(ones_v, cnt_sh.at[sd.at[p, b, 1]],
                                ssem[p]).wait()

    pltpu.sync_copy(z16_hbm.at[pl.ds(row0, RPT)], acc_sh.at[pl.ds(row0, RPT)])
    if with_cnt:
      pltpu.sync_copy(z1_hbm.at[pl.ds(row0, RPT)], cnt_sh.at[pl.ds(row0, RPT)])
      for i in range(CHUNK // 16):
        ones_v[pl.ds(i * 16, 16)] = jnp.full((16,), 1.0, jnp.float32)
    plsc.subcore_barrier()

    stage(0, 0)
    fire_g(0)

    def group2(i, carry):
      g = 2 * i
      stage(1, g + 1)
      fire_g(1)
      wait_g(0)
      fire_s(0)
      wait_s(0)

      @pl.when(i < G1 // 2 - 1)
      def _():
        stage(0, g + 2)
        fire_g(0)

      wait_g(1)
      fire_s(1)
      wait_s(1)
      return carry

    lax.fori_loop(0, G1 // 2, group2, 0)
    plsc.subcore_barrier()

    pltpu.sync_copy(acc_sh.at[pl.ds(row0, RPT)],
                    out_hbm.at[c, pl.ds(row0, RPT)])
    if with_cnt:
      pltpu.sync_copy(cnt_sh.at[pl.ds(row0, RPT)],
                      cnt_hbm.at[c, pl.ds(row0, RPT)])

  return pl.kernel(body,
                   out_type=tuple(out_type) if with_cnt else out_type[0],
                   mesh=mesh, scratch_types=scratch,
                   compiler_params=pltpu.CompilerParams(
                       use_tc_tiling_on_sc=False))


_edge_agg_cnt = _make_edge_aggregate(True)
_edge_agg = _make_edge_aggregate(False)


def _lin_relu_body(x_ref, w_ref, b_ref, o_ref):
  o_ref[...] = jnp.maximum(
      jnp.dot(x_ref[...], w_ref[...], preferred_element_type=jnp.float32)
      + b_ref[...], 0.0)


def _lin_relu(x, wt, b):
  din = x.shape[1]
  return pl.pallas_call(
      _lin_relu_body,
      grid=(NPAD // BN,),
      in_specs=[pl.BlockSpec((BN, din), lambda i: (i, 0)),
                pl.BlockSpec((din, F), lambda i: (0, 0)),
                pl.BlockSpec((1, F), lambda i: (0, 0))],
      out_specs=pl.BlockSpec((BN, F), lambda i: (i, 0)),
      out_shape=jax.ShapeDtypeStruct((NPAD, F), jnp.float32),
  )(x, wt, b)


def _update_body(want_y2, want_out,
                 x_ref, s_ref, cnt_ref, wx_ref, wm_ref, ba_ref,
                 w2_ref, b2_ref, *out_refs):
  ssum = s_ref[0] + s_ref[1]
  cc = cnt_ref[0] + cnt_ref[1]
  mean = ssum / jnp.maximum(cc, 1.0)
  h = (jnp.dot(x_ref[...], wx_ref[...], preferred_element_type=jnp.float32)
       + jnp.dot(mean, wm_ref[...], preferred_element_type=jnp.float32)
       + ba_ref[...])
  h = jnp.maximum(h, 0.0)
  norm = jnp.sqrt(jnp.sum(h * h, axis=1, keepdims=True))
  h = h / jnp.maximum(norm, 1e-12)
  o = 0
  if want_y2:
    out_refs[o][...] = h
    o += 1
    out_refs[o][...] = jnp.maximum(
        jnp.dot(h, w2_ref[...], preferred_element_type=jnp.float32)
        + b2_ref[...], 0.0)
    o += 1
  if want_out:
    out_refs[o][...] = (
        jnp.dot(h, w2_ref[...], preferred_element_type=jnp.float32)
        + b2_ref[...])


def _sage_update(x, sums, cnt, w_agg, b_agg, w_next, b_next,
                 want_y2):
  din = x.shape[1]
  wx = w_agg.T[:din]
  wm = w_agg.T[din:]
  fo = w_next.shape[0]
  if want_y2:
    out_shape = (jax.ShapeDtypeStruct((NPAD, F), jnp.float32),
                 jax.ShapeDtypeStruct((NPAD, fo), jnp.float32))
    out_specs = (pl.BlockSpec((BN, F), lambda i: (i, 0)),
                 pl.BlockSpec((BN, fo), lambda i: (i, 0)))
  else:
    out_shape = jax.ShapeDtypeStruct((NPAD, fo), jnp.float32)
    out_specs = pl.BlockSpec((BN, fo), lambda i: (i, 0))
  return pl.pallas_call(
      functools.partial(_update_body, want_y2, not want_y2),
      grid=(NPAD // BN,),
      in_specs=[pl.BlockSpec((BN, din), lambda i: (i, 0)),
                pl.BlockSpec((2, BN, F), lambda i: (0, i, 0)),
                pl.BlockSpec((2, BN, 1), lambda i: (0, i, 0)),
                pl.BlockSpec((din, F), lambda i: (0, 0)),
                pl.BlockSpec((F, F), lambda i: (0, 0)),
                pl.BlockSpec((1, F), lambda i: (0, 0)),
                pl.BlockSpec((F, fo), lambda i: (0, 0)),
                pl.BlockSpec((1, fo), lambda i: (0, 0))],
      out_specs=out_specs,
      out_shape=out_shape,
  )(x, sums, cnt, wx, wm, b_agg[None, :], w_next.T, b_next[None, :])


def kernel(x, edge_index, W1_lin, b1_lin, W1_agg, b1_agg,
           W2_lin, b2_lin, W2_agg, b2_agg, W_out, b_out):
  n = x.shape[0]
  e = edge_index.shape[1]

  x_pad = jnp.pad(x.astype(jnp.float32), ((0, NPAD - n), (0, 0)))

  src = edge_index[0].astype(jnp.int32)
  dst = edge_index[1].astype(jnp.int32)
  pad_e = EPAD - e
  fill = jnp.full((pad_e,), n, jnp.int32)
  src_c = jnp.concatenate([src, fill]).reshape(TOTAL_CHUNKS, CHUNK)
  dst_c = jnp.concatenate([dst, fill]).reshape(TOTAL_CHUNKS, CHUNK)
  sd = jnp.stack([src_c, dst_c], axis=1)

  z16 = jnp.zeros((NPAD, F), jnp.float32)
  z1 = jnp.zeros((NPAD,), jnp.float32)

  y1 = _lin_relu(x_pad, W1_lin.T, b1_lin[None, :])
  s1, cnt = _edge_agg_cnt(y1, sd, z16, z1)
  cnt3 = cnt.reshape(2, NPAD, 1)
  h1, y2 = _sage_update(x_pad, s1, cnt3, W1_agg, b1_agg, W2_lin, b2_lin,
                        want_y2=True)

  s2 = _edge_agg(y2, sd, z16)
  out = _sage_update(h1, s2, cnt3, W2_agg, b2_agg, W_out, b_out,
                     want_y2=False)
  return out[:n]
